# baseline (device time: 35244 ns/iter reference)
import jax
import jax.numpy as jnp
from jax import lax
from jax.experimental import pallas as pl
from jax.experimental.pallas import tpu as pltpu


N_CHUNKS = 8


def kernel(x):
    m, n = x.shape
    rows = m // N_CHUNKS

    def body(x_hbm, out_hbm, xloc, xbuf, sbuf, cp_sems, st_sems,
             sx_sems, rx_sems, sy_sems, ry_sems):
        my_x = lax.axis_index("x")
        my_y = lax.axis_index("y")

        barrier_sem = pltpu.get_barrier_semaphore()
        pl.semaphore_signal(
            barrier_sem, inc=1,
            device_id=(1 - my_x, my_y), device_id_type=pl.DeviceIdType.MESH,
        )
        pl.semaphore_signal(
            barrier_sem, inc=1,
            device_id=(my_x, 1 - my_y), device_id_type=pl.DeviceIdType.MESH,
        )
        pl.semaphore_wait(barrier_sem, 2)

        x_rdmas = []
        cps = []
        for c in range(N_CHUNKS):
            rs = pl.ds(c * rows, rows)
            r = pltpu.make_async_remote_copy(
                src_ref=x_hbm.at[rs, :],
                dst_ref=xbuf.at[rs, :],
                send_sem=sx_sems.at[c],
                recv_sem=rx_sems.at[c],
                device_id=(1 - my_x, my_y),
                device_id_type=pl.DeviceIdType.MESH,
            )
            r.start()
            x_rdmas.append(r)
            cp = pltpu.make_async_copy(x_hbm.at[rs, :], xloc.at[rs, :],
                                       cp_sems.at[c])
            cp.start()
            cps.append(cp)

        y_rdmas = []
        stores = []
        for c in range(N_CHUNKS):
            rs = pl.ds(c * rows, rows)
            cps[c].wait()
            x_rdmas[c].wait_recv()
            sbuf[rs, :] = xloc[rs, :] + xbuf[rs, :]
            st = pltpu.make_async_copy(
                sbuf.at[rs, :], out_hbm.at[rs, pl.ds(my_y * n, n)],
                st_sems.at[c])
            st.start()
            stores.append(st)
            r = pltpu.make_async_remote_copy(
                src_ref=sbuf.at[rs, :],
                dst_ref=out_hbm.at[rs, pl.ds(my_y * n, n)],
                send_sem=sy_sems.at[c],
                recv_sem=ry_sems.at[c],
                device_id=(my_x, 1 - my_y),
                device_id_type=pl.DeviceIdType.MESH,
            )
            r.start()
            y_rdmas.append(r)

        for c in range(N_CHUNKS):
            stores[c].wait()
            y_rdmas[c].wait_recv()
            x_rdmas[c].wait_send()
            y_rdmas[c].wait_send()

    out_shape = jax.ShapeDtypeStruct((m, 2 * n), jnp.float32)
    return pl.pallas_call(
        body,
        out_shape=out_shape,
        in_specs=[pl.BlockSpec(memory_space=pl.ANY)],
        out_specs=pl.BlockSpec(memory_space=pl.ANY),
        scratch_shapes=[
            pltpu.VMEM((m, n), jnp.float32),
            pltpu.VMEM((m, n), jnp.float32),
            pltpu.VMEM((m, n), jnp.float32),
            pltpu.SemaphoreType.DMA((N_CHUNKS,)),
            pltpu.SemaphoreType.DMA((N_CHUNKS,)),
            pltpu.SemaphoreType.DMA((N_CHUNKS,)),
            pltpu.SemaphoreType.DMA((N_CHUNKS,)),
            pltpu.SemaphoreType.DMA((N_CHUNKS,)),
            pltpu.SemaphoreType.DMA((N_CHUNKS,)),
        ],
        compiler_params=pltpu.CompilerParams(collective_id=0),
    )(x)
